# baseline (speedup 1.0000x reference)
import functools

import jax
import jax.numpy as jnp
import numpy as np
from jax import lax
from jax.experimental import pallas as pl
from jax.experimental.pallas import tpu as pltpu
from jax.experimental.pallas import tpu_sc as plsc

PH = 7
PW = 7
SCALE = 64.0
H = 128
W = 128
C = 256
N_ROIS = 1000

L = 16
NW = 32
NPAD = 1024
R_PER_W = NPAD // NW
ROW_IDX = PW * L
NBINS = PH * PW
BING = 4
BIN_ELEMS = NBINS * C
CHUNKS = C // L
NBUF = 3


def _permv(vec, idxvec):
    dnums = lax.GatherDimensionNumbers(
        offset_dims=(), collapsed_slice_dims=(0,), start_index_map=(0,))
    return lax.gather(vec, idxvec[:, None], dnums, (1,),
                      mode=lax.GatherScatterMode.PROMISE_IN_BOUNDS)


def _splat_lane(vec, lane):
    return _permv(vec, jnp.full((L,), lane, jnp.int32))


def _transpose16(vs, lane):
    for s in (1, 2, 4, 8):
        idxv = lane ^ s
        m = (lane & s) != 0
        new = [None] * L
        for i in range(L):
            p = _permv(vs[i ^ s], idxv)
            if i & s == 0:
                new[i] = jnp.where(m, p, vs[i])
            else:
                new[i] = jnp.where(m, vs[i], p)
        vs = new
    return vs


def _make_sc_kernel():
    mesh = plsc.VectorSubcoreMesh(core_axis_name="c", subcore_axis_name="s")

    @functools.partial(
        pl.kernel,
        out_type=jax.ShapeDtypeStruct((NPAD * BIN_ELEMS,), jnp.float32),
        mesh=mesh,
        scratch_types=[
            pltpu.VMEM((R_PER_W * 4 + L,), jnp.float32),
            pltpu.VMEM((ROW_IDX,), jnp.int32),
            pltpu.VMEM((ROW_IDX,), jnp.int32),
            pltpu.VMEM((ROW_IDX,), jnp.int32),
            pltpu.VMEM((ROW_IDX,), jnp.float32),
            pltpu.VMEM((ROW_IDX,), jnp.float32),
            pltpu.VMEM((ROW_IDX,), jnp.float32),
            pltpu.VMEM((ROW_IDX, C), jnp.float32),
            pltpu.VMEM((ROW_IDX, C), jnp.float32),
            pltpu.VMEM((ROW_IDX, C), jnp.float32),
            pltpu.VMEM((BING * L * C,), jnp.float32),
            pltpu.VMEM((C * NBINS + L,), jnp.float32),
            pltpu.SemaphoreType.DMA,
            pltpu.SemaphoreType.DMA,
            pltpu.SemaphoreType.DMA,
        ],
    )
    def roialign(fmap_hbm, rois_hbm, out_hbm, rois_v, idx0, idx1, idx2,
                 w0, w1, w2, rows0, rows1, rows2, stage1, stage2,
                 sem0, sem1, sem2):
        wid = lax.axis_index("c") * 16 + lax.axis_index("s")
        idx_bufs = (idx0, idx1, idx2)
        w_bufs = (w0, w1, w2)
        row_bufs = (rows0, rows1, rows2)
        sems = (sem0, sem1, sem2)

        pltpu.sync_copy(rois_hbm.at[pl.ds(wid * R_PER_W * 4, R_PER_W * 4)],
                        rois_v.at[pl.ds(0, R_PER_W * 4)])

        lane = lax.iota(jnp.int32, L)
        lane_syf = ((lane >> 3) & 1).astype(jnp.float32)
        lane_sxf = ((lane >> 2) & 1).astype(jnp.float32)
        lane_cy0 = ((lane >> 1) & 1) == 0
        lane_cx0 = (lane & 1) == 0

        def roi_body(r, carry):
            roi_chunk = rois_v[pl.ds(r * 4, L)]

            def splat(comp):
                return _splat_lane(roi_chunk, comp)

            x1s = splat(0) * SCALE
            y1s = splat(1) * SCALE
            x2s = splat(2) * SCALE
            y2s = splat(3) * SCALE
            roi_w = jnp.maximum(x2s - x1s, 1.0)
            roi_h = jnp.maximum(y2s - y1s, 1.0)
            bin_w = roi_w / float(PW)
            bin_h = roi_h / float(PH)

            def fill_row(ph):
                b = ph % NBUF
                ph_f = float(ph)

                def pw_body(pw, _):
                    pw_f = pw.astype(jnp.float32)
                    y = y1s + (ph_f + 0.25 + 0.5 * lane_syf) * bin_h
                    x = x1s + (pw_f + 0.25 + 0.5 * lane_sxf) * bin_w
                    valid = ((y > -1.0) & (y < float(H)) &
                             (x > -1.0) & (x < float(W)))
                    yc = jnp.minimum(jnp.maximum(y, 0.0), float(H - 1))
                    xc = jnp.minimum(jnp.maximum(x, 0.0), float(W - 1))
                    ylo = yc.astype(jnp.int32)
                    xlo = xc.astype(jnp.int32)
                    yhi = jnp.minimum(ylo + 1, H - 1)
                    xhi = jnp.minimum(xlo + 1, W - 1)
                    ly = yc - ylo.astype(jnp.float32)
                    lx = xc - xlo.astype(jnp.float32)
                    wy = jnp.where(lane_cy0, 1.0 - ly, ly)
                    wx = jnp.where(lane_cx0, 1.0 - lx, lx)
                    yi = jnp.where(lane_cy0, ylo, yhi)
                    xi = jnp.where(lane_cx0, xlo, xhi)
                    idx_bufs[b][pl.ds(pw * L, L)] = yi * W + xi
                    w_bufs[b][pl.ds(pw * L, L)] = (
                        wy * wx * jnp.where(valid, 0.25, 0.0))
                    return 0

                lax.fori_loop(0, PW, pw_body, 0)
                return pltpu.async_copy(fmap_hbm.at[idx_bufs[b]], row_bufs[b],
                                        sems[b])

            def combine_row(ph):
                b = ph % NBUF
                rows = row_bufs[b]
                wref = w_bufs[b]

                def pw_body(pw, _):
                    jbase = pw * L
                    wvec = wref[pl.ds(jbase, L)]

                    def j_body(j, acc):
                        wj = _splat_lane(wvec, j)
                        return tuple(
                            acc[k] + wj * rows[jbase + j, pl.ds(k * L, L)]
                            for k in range(CHUNKS))

                    acc = lax.fori_loop(
                        0, L, j_body,
                        tuple(jnp.zeros((L,), jnp.float32)
                              for _ in range(CHUNKS)))
                    obase = (ph * PW + pw) * C
                    for k in range(CHUNKS):
                        stage1[pl.ds(obase + k * L, L)] = acc[k]
                    return 0

                lax.fori_loop(0, PW, pw_body, 0)

            cps = [None] * NBUF
            for ph in range(PH):
                cps[ph % NBUF] = fill_row(ph)
                if ph >= NBUF - 1:
                    cps[(ph - NBUF + 1) % NBUF].wait()
                    combine_row(ph - NBUF + 1)
            for ph in range(PH - NBUF + 1, PH):
                cps[ph % NBUF].wait()
                combine_row(ph)

            def k_body(k, _):
                for g in (3, 0, 1, 2):
                    vs = [stage1[pl.ds((g * L + b) * C + k * L, L)]
                          for b in range(L)]
                    ws = _transpose16(vs, lane)
                    for c in range(L):
                        stage2[pl.ds(k * (L * NBINS) + c * NBINS + g * L, L)
                               ] = ws[c]
                return 0

            lax.fori_loop(0, CHUNKS, k_body, 0)

            out_base = (wid * R_PER_W + r) * BIN_ELEMS
            pltpu.sync_copy(stage2.at[pl.ds(0, BIN_ELEMS)],
                            out_hbm.at[pl.ds(out_base, BIN_ELEMS)])
            return carry

        lax.fori_loop(0, R_PER_W, roi_body, 0)

    return roialign


_SC_KERNEL = _make_sc_kernel()


@jax.jit
def kernel(features, rois):
    fmap = jnp.transpose(features, (0, 2, 3, 1)).reshape(H * W, C)
    rois_p = jnp.pad(rois, ((0, NPAD - N_ROIS), (0, 0))).reshape(NPAD * 4)
    out_flat = _SC_KERNEL(fmap, rois_p)
    out = out_flat[:N_ROIS * BIN_ELEMS].reshape(N_ROIS, C, PH, PW)
    return out


# speedup vs baseline: 1.5445x; 1.5445x over previous
import functools

import jax
import jax.numpy as jnp
import numpy as np
from jax import lax
from jax.experimental import pallas as pl
from jax.experimental.pallas import tpu as pltpu
from jax.experimental.pallas import tpu_sc as plsc

PH = 7
PW = 7
SCALE = 64.0
H = 128
W = 128
C = 256
N_ROIS = 1000

L = 16
NW = 32
NPAD = 1024
R_PER_W = NPAD // NW
ROW_IDX = PW * L
NBINS = PH * PW
BING = 4
BIN_ELEMS = NBINS * C
CHUNKS = C // L
NBUF = 3


def _permv(vec, idxvec):
    dnums = lax.GatherDimensionNumbers(
        offset_dims=(), collapsed_slice_dims=(0,), start_index_map=(0,))
    return lax.gather(vec, idxvec[:, None], dnums, (1,),
                      mode=lax.GatherScatterMode.PROMISE_IN_BOUNDS)


def _splat_lane(vec, lane):
    return _permv(vec, jnp.full((L,), lane, jnp.int32))


def _make_sc_kernel():
    mesh = plsc.VectorSubcoreMesh(core_axis_name="c", subcore_axis_name="s")

    @functools.partial(
        pl.kernel,
        out_type=jax.ShapeDtypeStruct((NPAD * BIN_ELEMS,), jnp.float32),
        mesh=mesh,
        scratch_types=[
            pltpu.VMEM((R_PER_W * 4 + L,), jnp.float32),
            pltpu.VMEM((ROW_IDX,), jnp.int32),
            pltpu.VMEM((ROW_IDX,), jnp.int32),
            pltpu.VMEM((ROW_IDX,), jnp.int32),
            pltpu.VMEM((ROW_IDX,), jnp.float32),
            pltpu.VMEM((ROW_IDX,), jnp.float32),
            pltpu.VMEM((ROW_IDX,), jnp.float32),
            pltpu.VMEM((ROW_IDX, C), jnp.float32),
            pltpu.VMEM((ROW_IDX, C), jnp.float32),
            pltpu.VMEM((ROW_IDX, C), jnp.float32),
            pltpu.VMEM((BIN_ELEMS,), jnp.float32),
            pltpu.SemaphoreType.DMA,
            pltpu.SemaphoreType.DMA,
            pltpu.SemaphoreType.DMA,
        ],
    )
    def roialign(fmap_hbm, rois_hbm, out_hbm, rois_v, idx0, idx1, idx2,
                 w0, w1, w2, rows0, rows1, rows2, stage1,
                 sem0, sem1, sem2):
        wid = lax.axis_index("c") * 16 + lax.axis_index("s")
        idx_bufs = (idx0, idx1, idx2)
        w_bufs = (w0, w1, w2)
        row_bufs = (rows0, rows1, rows2)
        sems = (sem0, sem1, sem2)

        pltpu.sync_copy(rois_hbm.at[pl.ds(wid * R_PER_W * 4, R_PER_W * 4)],
                        rois_v.at[pl.ds(0, R_PER_W * 4)])

        lane = lax.iota(jnp.int32, L)
        lane_syf = ((lane >> 3) & 1).astype(jnp.float32)
        lane_sxf = ((lane >> 2) & 1).astype(jnp.float32)
        lane_cy0 = ((lane >> 1) & 1) == 0
        lane_cx0 = (lane & 1) == 0

        def roi_body(r, carry):
            roi_chunk = rois_v[pl.ds(r * 4, L)]

            def splat(comp):
                return _splat_lane(roi_chunk, comp)

            x1s = splat(0) * SCALE
            y1s = splat(1) * SCALE
            x2s = splat(2) * SCALE
            y2s = splat(3) * SCALE
            roi_w = jnp.maximum(x2s - x1s, 1.0)
            roi_h = jnp.maximum(y2s - y1s, 1.0)
            bin_w = roi_w / float(PW)
            bin_h = roi_h / float(PH)

            def fill_row(ph):
                b = ph % NBUF
                ph_f = float(ph)

                def pw_body(pw, _):
                    pw_f = pw.astype(jnp.float32)
                    y = y1s + (ph_f + 0.25 + 0.5 * lane_syf) * bin_h
                    x = x1s + (pw_f + 0.25 + 0.5 * lane_sxf) * bin_w
                    valid = ((y > -1.0) & (y < float(H)) &
                             (x > -1.0) & (x < float(W)))
                    yc = jnp.minimum(jnp.maximum(y, 0.0), float(H - 1))
                    xc = jnp.minimum(jnp.maximum(x, 0.0), float(W - 1))
                    ylo = yc.astype(jnp.int32)
                    xlo = xc.astype(jnp.int32)
                    yhi = jnp.minimum(ylo + 1, H - 1)
                    xhi = jnp.minimum(xlo + 1, W - 1)
                    ly = yc - ylo.astype(jnp.float32)
                    lx = xc - xlo.astype(jnp.float32)
                    wy = jnp.where(lane_cy0, 1.0 - ly, ly)
                    wx = jnp.where(lane_cx0, 1.0 - lx, lx)
                    yi = jnp.where(lane_cy0, ylo, yhi)
                    xi = jnp.where(lane_cx0, xlo, xhi)
                    idx_bufs[b][pl.ds(pw * L, L)] = yi * W + xi
                    w_bufs[b][pl.ds(pw * L, L)] = (
                        wy * wx * jnp.where(valid, 0.25, 0.0))
                    return 0

                lax.fori_loop(0, PW, pw_body, 0)
                return pltpu.async_copy(fmap_hbm.at[idx_bufs[b]], row_bufs[b],
                                        sems[b])

            def combine_row(ph):
                b = ph % NBUF
                rows = row_bufs[b]
                wref = w_bufs[b]

                def pw_body(pw, _):
                    jbase = pw * L
                    wvec = wref[pl.ds(jbase, L)]

                    def j_body(j, acc):
                        wj = _splat_lane(wvec, j)
                        return tuple(
                            acc[k] + wj * rows[jbase + j, pl.ds(k * L, L)]
                            for k in range(CHUNKS))

                    acc = lax.fori_loop(
                        0, L, j_body,
                        tuple(jnp.zeros((L,), jnp.float32)
                              for _ in range(CHUNKS)))
                    obase = (ph * PW + pw) * C
                    for k in range(CHUNKS):
                        stage1[pl.ds(obase + k * L, L)] = acc[k]
                    return 0

                lax.fori_loop(0, PW, pw_body, 0)

            cps = [None] * NBUF
            for ph in range(PH):
                cps[ph % NBUF] = fill_row(ph)
                if ph >= NBUF - 1:
                    cps[(ph - NBUF + 1) % NBUF].wait()
                    combine_row(ph - NBUF + 1)
            for ph in range(PH - NBUF + 1, PH):
                cps[ph % NBUF].wait()
                combine_row(ph)

            out_base = (wid * R_PER_W + r) * BIN_ELEMS
            pltpu.sync_copy(stage1,
                            out_hbm.at[pl.ds(out_base, BIN_ELEMS)])
            return carry

        lax.fori_loop(0, R_PER_W, roi_body, 0)

    return roialign


_SC_KERNEL = _make_sc_kernel()

TR_BLK = 40


def _tr_body(in_ref, out_ref):
    out_ref[...] = jnp.swapaxes(in_ref[...], 1, 2)


_TC_TRANSPOSE = pl.pallas_call(
    _tr_body,
    grid=(N_ROIS // TR_BLK,),
    in_specs=[pl.BlockSpec((TR_BLK, NBINS, C), lambda i: (i, 0, 0))],
    out_specs=pl.BlockSpec((TR_BLK, C, NBINS), lambda i: (i, 0, 0)),
    out_shape=jax.ShapeDtypeStruct((N_ROIS, C, NBINS), jnp.float32),
)


@jax.jit
def kernel(features, rois):
    fmap = jnp.transpose(features, (0, 2, 3, 1)).reshape(H * W, C)
    rois_p = jnp.pad(rois, ((0, NPAD - N_ROIS), (0, 0))).reshape(NPAD * 4)
    out_flat = _SC_KERNEL(fmap, rois_p)
    out_nhwc = out_flat[:N_ROIS * BIN_ELEMS].reshape(N_ROIS, NBINS, C)
    out = _TC_TRANSPOSE(out_nhwc)
    return out.reshape(N_ROIS, C, PH, PW)
